# baseline (speedup 1.0000x reference)
import functools

import jax
import jax.numpy as jnp
from jax import lax
from jax.experimental import pallas as pl
from jax.experimental.pallas import tpu as pltpu
from jax.experimental.pallas import tpu_sc as plsc

N_USERS = 100000
N_ITEMS = 1000000
K = 64
BATCH = 16384

NC = 2
NS = 16
L = 16
NW = NC * NS
BPW = BATCH // NW
CHUNK = 128
NCHUNK = BPW // CHUNK


def _fpmc_body(uid_hbm, bp_hbm, iid_hbm, vil_hbm, vli_hbm, vui_hbm, viu_hbm,
               out_hbm,
               uid_v, bp_v, iid_v, a0_v, b0_v, a1_v, b1_v, out_v, tr_v, sem):
  wid = lax.axis_index("s") * NC + lax.axis_index("c")

  pltpu.sync_copy(uid_hbm.at[wid], uid_v)
  pltpu.sync_copy(bp_hbm.at[wid], bp_v)
  pltpu.sync_copy(iid_hbm.at[wid], iid_v)

  for j in range(NCHUNK):
    d0 = pltpu.async_copy(vui_hbm.at[uid_v.at[j]], a0_v, sem)
    d1 = pltpu.async_copy(viu_hbm.at[iid_v.at[j]], b0_v, sem)
    d2 = pltpu.async_copy(vil_hbm.at[iid_v.at[j]], a1_v, sem)
    d3 = pltpu.async_copy(vli_hbm.at[bp_v.at[j]], b1_v, sem)
    d0.wait()
    d1.wait()
    d2.wait()
    d3.wait()

    @pl.loop(0, CHUNK // L)
    def _compute(c):
      lane = lax.iota(jnp.int32, L)
      for r in range(L):
        i = c * L + r
        s = a0_v[i, pl.ds(0, L)] * b0_v[i, pl.ds(0, L)]
        s += a1_v[i, pl.ds(0, L)] * b1_v[i, pl.ds(0, L)]
        for g in range(1, K // L):
          s += a0_v[i, pl.ds(g * L, L)] * b0_v[i, pl.ds(g * L, L)]
          s += a1_v[i, pl.ds(g * L, L)] * b1_v[i, pl.ds(g * L, L)]
        plsc.store_scatter(tr_v, [lane * L + r], s)
      acc = tr_v[pl.ds(0, L)]
      for l in range(1, L):
        acc += tr_v[pl.ds(l * L, L)]
      out_v[pl.ds(j * CHUNK + c * L, L)] = acc

  pltpu.sync_copy(out_v, out_hbm.at[pl.ds(wid * BPW, BPW)])


@jax.jit
def _fpmc_sc(uid, basket_prev, iid, VIL, VLI, VUI, VIU):
  mesh = plsc.VectorSubcoreMesh(
      core_axis_name="c", subcore_axis_name="s", num_cores=NC, num_subcores=NS)
  run = pl.kernel(
      _fpmc_body,
      out_type=jax.ShapeDtypeStruct((BATCH,), jnp.float32),
      mesh=mesh,
      compiler_params=pltpu.CompilerParams(
          needs_layout_passes=False, use_tc_tiling_on_sc=False),
      scratch_types=[
          pltpu.VMEM((NCHUNK, CHUNK), jnp.int32),
          pltpu.VMEM((NCHUNK, CHUNK), jnp.int32),
          pltpu.VMEM((NCHUNK, CHUNK), jnp.int32),
          pltpu.VMEM((CHUNK, K), jnp.float32),
          pltpu.VMEM((CHUNK, K), jnp.float32),
          pltpu.VMEM((CHUNK, K), jnp.float32),
          pltpu.VMEM((CHUNK, K), jnp.float32),
          pltpu.VMEM((BPW,), jnp.float32),
          pltpu.VMEM((L * L,), jnp.float32),
          pltpu.SemaphoreType.DMA,
      ],
  )
  uid3 = uid.astype(jnp.int32).reshape(NW, NCHUNK, CHUNK)
  bp3 = basket_prev.astype(jnp.int32).reshape(NW, NCHUNK, CHUNK)
  iid3 = iid.astype(jnp.int32).reshape(NW, NCHUNK, CHUNK)
  return run(uid3, bp3, iid3, VIL, VLI, VUI, VIU)


def kernel(uid, basket_prev, iid, VIL, VLI, VUI, VIU):
  return _fpmc_sc(uid, basket_prev, iid, VIL, VLI, VUI, VIU)


# speedup vs baseline: 1.0031x; 1.0031x over previous
import functools

import jax
import jax.numpy as jnp
from jax import lax
from jax.experimental import pallas as pl
from jax.experimental.pallas import tpu as pltpu
from jax.experimental.pallas import tpu_sc as plsc

N_USERS = 100000
N_ITEMS = 1000000
K = 64
BATCH = 16384

NC = 2
NS = 16
L = 16
NW = NC * NS
BPW = BATCH // NW
CHUNK = 128
NCHUNK = BPW // CHUNK


def _fpmc_body(uid_hbm, bp_hbm, iid_hbm, vil_hbm, vli_hbm, vui_hbm, viu_hbm,
               out_hbm,
               uid_v, bp_v, iid_v, rows_v, out_v, tr_v, sem0, sem1):
  wid = lax.axis_index("s") * NC + lax.axis_index("c")

  pltpu.sync_copy(uid_hbm.at[wid], uid_v)
  pltpu.sync_copy(bp_hbm.at[wid], bp_v)
  pltpu.sync_copy(iid_hbm.at[wid], iid_v)

  sems = [sem0, sem1]

  def fire(j):
    buf = rows_v.at[j % 2]
    sem = sems[j % 2]
    return [
        pltpu.async_copy(vui_hbm.at[uid_v.at[j]], buf.at[0], sem),
        pltpu.async_copy(viu_hbm.at[iid_v.at[j]], buf.at[1], sem),
        pltpu.async_copy(vil_hbm.at[iid_v.at[j]], buf.at[2], sem),
        pltpu.async_copy(vli_hbm.at[bp_v.at[j]], buf.at[3], sem),
    ]

  descs = fire(0)
  for j in range(NCHUNK):
    nxt = fire(j + 1) if j + 1 < NCHUNK else None
    for d in descs:
      d.wait()
    buf = rows_v.at[j % 2]
    a0_v, b0_v, a1_v, b1_v = buf.at[0], buf.at[1], buf.at[2], buf.at[3]

    @pl.loop(0, CHUNK // L)
    def _compute(c):
      lane = lax.iota(jnp.int32, L)
      for r in range(L):
        i = c * L + r
        s = a0_v[i, pl.ds(0, L)] * b0_v[i, pl.ds(0, L)]
        s += a1_v[i, pl.ds(0, L)] * b1_v[i, pl.ds(0, L)]
        for g in range(1, K // L):
          s += a0_v[i, pl.ds(g * L, L)] * b0_v[i, pl.ds(g * L, L)]
          s += a1_v[i, pl.ds(g * L, L)] * b1_v[i, pl.ds(g * L, L)]
        plsc.store_scatter(tr_v, [lane * L + r], s)
      acc = tr_v[pl.ds(0, L)]
      for l in range(1, L):
        acc += tr_v[pl.ds(l * L, L)]
      out_v[pl.ds(j * CHUNK + c * L, L)] = acc

    descs = nxt

  pltpu.sync_copy(out_v, out_hbm.at[pl.ds(wid * BPW, BPW)])


@jax.jit
def _fpmc_sc(uid, basket_prev, iid, VIL, VLI, VUI, VIU):
  mesh = plsc.VectorSubcoreMesh(
      core_axis_name="c", subcore_axis_name="s", num_cores=NC, num_subcores=NS)
  run = pl.kernel(
      _fpmc_body,
      out_type=jax.ShapeDtypeStruct((BATCH,), jnp.float32),
      mesh=mesh,
      compiler_params=pltpu.CompilerParams(
          needs_layout_passes=False, use_tc_tiling_on_sc=False),
      scratch_types=[
          pltpu.VMEM((NCHUNK, CHUNK), jnp.int32),
          pltpu.VMEM((NCHUNK, CHUNK), jnp.int32),
          pltpu.VMEM((NCHUNK, CHUNK), jnp.int32),
          pltpu.VMEM((2, 4, CHUNK, K), jnp.float32),
          pltpu.VMEM((BPW,), jnp.float32),
          pltpu.VMEM((L * L,), jnp.float32),
          pltpu.SemaphoreType.DMA,
          pltpu.SemaphoreType.DMA,
      ],
  )
  uid3 = uid.astype(jnp.int32).reshape(NW, NCHUNK, CHUNK)
  bp3 = basket_prev.astype(jnp.int32).reshape(NW, NCHUNK, CHUNK)
  iid3 = iid.astype(jnp.int32).reshape(NW, NCHUNK, CHUNK)
  return run(uid3, bp3, iid3, VIL, VLI, VUI, VIU)


def kernel(uid, basket_prev, iid, VIL, VLI, VUI, VIU):
  return _fpmc_sc(uid, basket_prev, iid, VIL, VLI, VUI, VIU)


# speedup vs baseline: 1.4767x; 1.4721x over previous
import functools

import jax
import jax.numpy as jnp
from jax import lax
from jax.experimental import pallas as pl
from jax.experimental.pallas import tpu as pltpu
from jax.experimental.pallas import tpu_sc as plsc

N_USERS = 100000
N_ITEMS = 1000000
K = 64
BATCH = 16384

NC = 2
NS = 16
L = 16
NW = NC * NS
BPW = BATCH // NW
G = 16
NG = BPW // G


def _fpmc_body(uid_hbm, bp_hbm, iid_hbm, vil_hbm, vli_hbm, vui_hbm, viu_hbm,
               out_hbm,
               uid_v, bp_v, iid_v, rows_v, out_v, tr_v, sem):
  wid = lax.axis_index("s") * NC + lax.axis_index("c")

  pltpu.sync_copy(uid_hbm.at[wid], uid_v)
  pltpu.sync_copy(bp_hbm.at[wid], bp_v)
  pltpu.sync_copy(iid_hbm.at[wid], iid_v)

  @pl.loop(0, NG)
  def _group(g):
    uvec = uid_v[pl.ds(g * G, L)]
    ivec = iid_v[pl.ds(g * G, L)]
    bvec = bp_v[pl.ds(g * G, L)]
    descs = []
    for r in range(G):
      u = uvec[r]
      it = ivec[r]
      bp = bvec[r]
      descs.append(pltpu.async_copy(vui_hbm.at[u], rows_v.at[r, 0], sem))
      descs.append(pltpu.async_copy(viu_hbm.at[it], rows_v.at[r, 1], sem))
      descs.append(pltpu.async_copy(vil_hbm.at[it], rows_v.at[r, 2], sem))
      descs.append(pltpu.async_copy(vli_hbm.at[bp], rows_v.at[r, 3], sem))
    for d in descs:
      d.wait()
    lane = lax.iota(jnp.int32, L)
    for r in range(G):
      s = rows_v[r, 0, pl.ds(0, L)] * rows_v[r, 1, pl.ds(0, L)]
      s += rows_v[r, 2, pl.ds(0, L)] * rows_v[r, 3, pl.ds(0, L)]
      for q in range(1, K // L):
        s += rows_v[r, 0, pl.ds(q * L, L)] * rows_v[r, 1, pl.ds(q * L, L)]
        s += rows_v[r, 2, pl.ds(q * L, L)] * rows_v[r, 3, pl.ds(q * L, L)]
      plsc.store_scatter(tr_v, [lane * L + r], s)
    acc = tr_v[pl.ds(0, L)]
    for l in range(1, L):
      acc += tr_v[pl.ds(l * L, L)]
    out_v[pl.ds(g * G, L)] = acc

  pltpu.sync_copy(out_v, out_hbm.at[pl.ds(wid * BPW, BPW)])


@jax.jit
def _fpmc_sc(uid, basket_prev, iid, VIL, VLI, VUI, VIU):
  mesh = plsc.VectorSubcoreMesh(
      core_axis_name="c", subcore_axis_name="s", num_cores=NC, num_subcores=NS)
  run = pl.kernel(
      _fpmc_body,
      out_type=jax.ShapeDtypeStruct((BATCH,), jnp.float32),
      mesh=mesh,
      compiler_params=pltpu.CompilerParams(
          needs_layout_passes=False, use_tc_tiling_on_sc=True),
      scratch_types=[
          pltpu.VMEM((BPW,), jnp.int32),
          pltpu.VMEM((BPW,), jnp.int32),
          pltpu.VMEM((BPW,), jnp.int32),
          pltpu.VMEM((G, 4, K), jnp.float32),
          pltpu.VMEM((BPW,), jnp.float32),
          pltpu.VMEM((L * L,), jnp.float32),
          pltpu.SemaphoreType.DMA,
      ],
  )
  uid2 = uid.astype(jnp.int32).reshape(NW, BPW)
  bp2 = basket_prev.astype(jnp.int32).reshape(NW, BPW)
  iid2 = iid.astype(jnp.int32).reshape(NW, BPW)
  return run(uid2, bp2, iid2, VIL, VLI, VUI, VIU)


def kernel(uid, basket_prev, iid, VIL, VLI, VUI, VIU):
  return _fpmc_sc(uid, basket_prev, iid, VIL, VLI, VUI, VIU)


# speedup vs baseline: 1.7873x; 1.2103x over previous
import functools

import jax
import jax.numpy as jnp
from jax import lax
from jax.experimental import pallas as pl
from jax.experimental.pallas import tpu as pltpu
from jax.experimental.pallas import tpu_sc as plsc

N_USERS = 100000
N_ITEMS = 1000000
K = 64
BATCH = 16384

NC = 2
NS = 16
L = 16
NW = NC * NS
BPW = BATCH // NW
G = 16
NG = BPW // G
TL = 128


NB = 3
BLK = 48


def _fpmc_body(uid_hbm, bp_hbm, iid_hbm, vil_hbm, vli_hbm, vui_hbm, viu_hbm,
               out_hbm,
               uid_v, bp_v, iid_v, blk_v, out_v, tr_v, sem0, sem1, sem2):
  wid = lax.axis_index("s") * NC + lax.axis_index("c")

  pltpu.sync_copy(uid_hbm.at[wid], uid_v)
  pltpu.sync_copy(bp_hbm.at[wid], bp_v)
  pltpu.sync_copy(iid_hbm.at[wid], iid_v)

  sems = [sem0, sem1, sem2]
  kiota = lax.iota(jnp.int32, L)

  def load_vecs(e):
    e = pl.multiple_of(e, L)
    u = uid_v[pl.ds(e, L)]
    i = iid_v[pl.ds(e, L)]
    b = bp_v[pl.ds(e, L)]
    return (u & ~(TL - 1), i & ~(TL - 1), b & ~(TL - 1),
            u & (TL - 1), i & (TL - 1), b & (TL - 1))

  def fire(gv, lane, phase):
    ubase, ibase, bbase = gv[0], gv[1], gv[2]
    buf = blk_v.at[phase]
    ub = pl.multiple_of(ubase[lane], TL)
    ib = pl.multiple_of(ibase[lane], TL)
    bb = pl.multiple_of(bbase[lane], TL)
    return [
        pltpu.async_copy(vui_hbm.at[:, pl.ds(ub, TL)], buf.at[0], sems[phase]),
        pltpu.async_copy(viu_hbm.at[:, pl.ds(ib, TL)], buf.at[1], sems[phase]),
        pltpu.async_copy(vil_hbm.at[:, pl.ds(ib, TL)], buf.at[2], sems[phase]),
        pltpu.async_copy(vli_hbm.at[:, pl.ds(bb, TL)], buf.at[3], sems[phase]),
    ]

  def compute(gv, lane, phase):
    uoff, ioff, boff = gv[3], gv[4], gv[5]
    buf = blk_v.at[phase]
    ci = jnp.full((L,), uoff[lane], jnp.int32)
    cj = jnp.full((L,), ioff[lane], jnp.int32)
    ck = jnp.full((L,), boff[lane], jnp.int32)
    s = None
    for q in range(K // L):
      rows = q * L + kiota
      a0 = plsc.load_gather(buf.at[0], [rows, ci])
      b0 = plsc.load_gather(buf.at[1], [rows, cj])
      a1 = plsc.load_gather(buf.at[2], [rows, cj])
      b1 = plsc.load_gather(buf.at[3], [rows, ck])
      p = a0 * b0 + a1 * b1
      s = p if s is None else s + p
    plsc.store_scatter(tr_v, [kiota * L + lane], s)

  def do_block(e0, nj):
    gv = load_vecs(e0)
    descs = fire(gv, 0, 0)
    for j in range(nj):
      if j + 1 < nj:
        if (j + 1) % G == 0:
          gv_next = load_vecs(e0 + j + 1)
        else:
          gv_next = gv
        nd = fire(gv_next, (j + 1) % G, (j + 1) % NB)
      else:
        gv_next, nd = gv, None
      for d in descs:
        d.wait()
      compute(gv, j % G, j % NB)
      if j % G == G - 1:
        acc = tr_v[pl.ds(0, L)]
        for l in range(1, L):
          acc += tr_v[pl.ds(l * L, L)]
        out_v[pl.ds(e0 + j - (G - 1), L)] = acc
      gv = gv_next
      descs = nd

  @pl.loop(0, (BPW // BLK) * BLK // BLK)
  def _blk(h):
    do_block(h * BLK, BLK)

  _tail = BPW - (BPW // BLK) * BLK
  if _tail:
    do_block((BPW // BLK) * BLK, _tail)

  pltpu.sync_copy(out_v, out_hbm.at[pl.ds(wid * BPW, BPW)])


@jax.jit
def _fpmc_sc(uid, basket_prev, iid, VIL, VLI, VUI, VIU):
  mesh = plsc.VectorSubcoreMesh(
      core_axis_name="c", subcore_axis_name="s", num_cores=NC, num_subcores=NS)
  run = pl.kernel(
      _fpmc_body,
      out_type=jax.ShapeDtypeStruct((BATCH,), jnp.float32),
      mesh=mesh,
      compiler_params=pltpu.CompilerParams(
          needs_layout_passes=False, use_tc_tiling_on_sc=True),
      scratch_types=[
          pltpu.VMEM((BPW,), jnp.int32),
          pltpu.VMEM((BPW,), jnp.int32),
          pltpu.VMEM((BPW,), jnp.int32),
          pltpu.VMEM((NB, 4, K, TL), jnp.float32),
          pltpu.VMEM((BPW,), jnp.float32),
          pltpu.VMEM((L * L,), jnp.float32),
          pltpu.SemaphoreType.DMA,
          pltpu.SemaphoreType.DMA,
          pltpu.SemaphoreType.DMA,
      ],
  )
  uid2 = uid.astype(jnp.int32).reshape(NW, BPW)
  bp2 = basket_prev.astype(jnp.int32).reshape(NW, BPW)
  iid2 = iid.astype(jnp.int32).reshape(NW, BPW)
  return run(uid2, bp2, iid2, VIL.T, VLI.T, VUI.T, VIU.T)


def kernel(uid, basket_prev, iid, VIL, VLI, VUI, VIU):
  return _fpmc_sc(uid, basket_prev, iid, VIL, VLI, VUI, VIU)


# speedup vs baseline: 1.9693x; 1.1018x over previous
import functools

import jax
import jax.numpy as jnp
from jax import lax
from jax.experimental import pallas as pl
from jax.experimental.pallas import tpu as pltpu
from jax.experimental.pallas import tpu_sc as plsc

N_USERS = 100000
N_ITEMS = 1000000
K = 64
BATCH = 16384

NC = 2
NS = 16
L = 16
NW = NC * NS
BPW = BATCH // NW
G = 16
NG = BPW // G
TL = 128


NB = 6
DEPTH = 3
HK = K // 2
BLK = 48


def _fpmc_body(uid_hbm, bp_hbm, iid_hbm, vil_hbm, vli_hbm, vui_hbm, viu_hbm,
               out_hbm,
               uid_v, bp_v, iid_v, blk_v, out_v, tr_v,
               sem0, sem1, sem2, sem3, sem4, sem5):
  wid = lax.axis_index("s") * NC + lax.axis_index("c")

  pltpu.sync_copy(uid_hbm.at[wid], uid_v)
  pltpu.sync_copy(bp_hbm.at[wid], bp_v)
  pltpu.sync_copy(iid_hbm.at[wid], iid_v)

  sems = [sem0, sem1, sem2, sem3, sem4, sem5]
  kiota = lax.iota(jnp.int32, L)

  def load_vecs(e):
    e = pl.multiple_of(e, L)
    u = uid_v[pl.ds(e, L)]
    i = iid_v[pl.ds(e, L)]
    b = bp_v[pl.ds(e, L)]
    return (u & ~(TL - 1), i & ~(TL - 1), b & ~(TL - 1),
            u & (TL - 1), i & (TL - 1), b & (TL - 1))

  def fire_sub(gv, s):
    lane = (s // 2) % G
    half = s % 2
    phase = s % NB
    buf = blk_v.at[phase]
    ub = pl.multiple_of(gv[0][lane], TL)
    ib = pl.multiple_of(gv[1][lane], TL)
    bb = pl.multiple_of(gv[2][lane], TL)
    ks = pl.ds(half * HK, HK)
    return [
        pltpu.async_copy(vui_hbm.at[ks, pl.ds(ub, TL)], buf.at[0], sems[phase]),
        pltpu.async_copy(viu_hbm.at[ks, pl.ds(ib, TL)], buf.at[1], sems[phase]),
        pltpu.async_copy(vil_hbm.at[ks, pl.ds(ib, TL)], buf.at[2], sems[phase]),
        pltpu.async_copy(vli_hbm.at[ks, pl.ds(bb, TL)], buf.at[3], sems[phase]),
    ]

  def compute_half(gv, s):
    lane = (s // 2) % G
    phase = s % NB
    buf = blk_v.at[phase]
    ci = jnp.full((L,), gv[3][lane], jnp.int32)
    cj = jnp.full((L,), gv[4][lane], jnp.int32)
    ck = jnp.full((L,), gv[5][lane], jnp.int32)
    p = None
    for q in range(HK // L):
      rows = q * L + kiota
      a0 = plsc.load_gather(buf.at[0], [rows, ci])
      b0 = plsc.load_gather(buf.at[1], [rows, cj])
      a1 = plsc.load_gather(buf.at[2], [rows, cj])
      b1 = plsc.load_gather(buf.at[3], [rows, ck])
      t = a0 * b0 + a1 * b1
      p = t if p is None else p + t
    return p

  def do_block(e0, nj):
    ns = 2 * nj
    gv_by_group = {}

    def get_gv(j):
      g = j // G
      if g not in gv_by_group:
        gv_by_group[g] = load_vecs(e0 + g * G)
      return gv_by_group[g]

    dq = [fire_sub(get_gv(s // 2), s) for s in range(DEPTH)]
    acc_el = None
    for s in range(ns):
      sf = s + DEPTH
      nd = fire_sub(get_gv(sf // 2), sf) if sf < ns else None
      for d in dq.pop(0):
        d.wait()
      p = compute_half(get_gv(s // 2), s)
      if s % 2 == 0:
        acc_el = p
      else:
        j = s // 2
        plsc.store_scatter(tr_v, [kiota * L + (j % G)], acc_el + p)
        if j % G == G - 1:
          acc = tr_v[pl.ds(0, L)]
          for l in range(1, L):
            acc += tr_v[pl.ds(l * L, L)]
          out_v[pl.ds(e0 + j - (G - 1), L)] = acc
      dq.append(nd)

  @pl.loop(0, BPW // BLK)
  def _blk(h):
    do_block(h * BLK, BLK)

  _tail = BPW - (BPW // BLK) * BLK
  if _tail:
    do_block((BPW // BLK) * BLK, _tail)

  pltpu.sync_copy(out_v, out_hbm.at[pl.ds(wid * BPW, BPW)])


@jax.jit
def _fpmc_sc(uid, basket_prev, iid, VIL, VLI, VUI, VIU):
  mesh = plsc.VectorSubcoreMesh(
      core_axis_name="c", subcore_axis_name="s", num_cores=NC, num_subcores=NS)
  run = pl.kernel(
      _fpmc_body,
      out_type=jax.ShapeDtypeStruct((BATCH,), jnp.float32),
      mesh=mesh,
      compiler_params=pltpu.CompilerParams(
          needs_layout_passes=False, use_tc_tiling_on_sc=True),
      scratch_types=[
          pltpu.VMEM((BPW,), jnp.int32),
          pltpu.VMEM((BPW,), jnp.int32),
          pltpu.VMEM((BPW,), jnp.int32),
          pltpu.VMEM((NB, 4, HK, TL), jnp.float32),
          pltpu.VMEM((BPW,), jnp.float32),
          pltpu.VMEM((L * L,), jnp.float32),
          pltpu.SemaphoreType.DMA,
          pltpu.SemaphoreType.DMA,
          pltpu.SemaphoreType.DMA,
          pltpu.SemaphoreType.DMA,
          pltpu.SemaphoreType.DMA,
          pltpu.SemaphoreType.DMA,
      ],
  )
  uid2 = uid.astype(jnp.int32).reshape(NW, BPW)
  bp2 = basket_prev.astype(jnp.int32).reshape(NW, BPW)
  iid2 = iid.astype(jnp.int32).reshape(NW, BPW)
  return run(uid2, bp2, iid2, VIL.T, VLI.T, VUI.T, VIU.T)


def kernel(uid, basket_prev, iid, VIL, VLI, VUI, VIU):
  return _fpmc_sc(uid, basket_prev, iid, VIL, VLI, VUI, VIU)


# speedup vs baseline: 2.0451x; 1.0385x over previous
import functools

import jax
import jax.numpy as jnp
from jax import lax
from jax.experimental import pallas as pl
from jax.experimental.pallas import tpu as pltpu
from jax.experimental.pallas import tpu_sc as plsc

N_USERS = 100000
N_ITEMS = 1000000
K = 64
BATCH = 16384

NC = 2
NS = 16
L = 16
NW = NC * NS
BPW = BATCH // NW
G = 16
NG = BPW // G
TL = 128


NB = 6
DEPTH = 4
HK = K // 2
BLK = 48


def _fpmc_body(uid_hbm, bp_hbm, iid_hbm, vil_hbm, vli_hbm, vui_hbm, viu_hbm,
               out_hbm,
               uid_v, bp_v, iid_v, blk_v, out_v, tr_v,
               sem0, sem1, sem2, sem3, sem4, sem5):
  wid = lax.axis_index("s") * NC + lax.axis_index("c")

  pltpu.sync_copy(uid_hbm.at[wid], uid_v)
  pltpu.sync_copy(bp_hbm.at[wid], bp_v)
  pltpu.sync_copy(iid_hbm.at[wid], iid_v)

  sems = [sem0, sem1, sem2, sem3, sem4, sem5]
  kiota = lax.iota(jnp.int32, L)

  def load_vecs(e):
    e = pl.multiple_of(e, L)
    u = uid_v[pl.ds(e, L)]
    i = iid_v[pl.ds(e, L)]
    b = bp_v[pl.ds(e, L)]
    return (u & ~(TL - 1), i & ~(TL - 1), b & ~(TL - 1),
            u & (TL - 1), i & (TL - 1), b & (TL - 1))

  def fire_sub(gv, s):
    lane = (s // 2) % G
    half = s % 2
    phase = s % NB
    buf = blk_v.at[phase]
    ub = pl.multiple_of(gv[0][lane], TL)
    ib = pl.multiple_of(gv[1][lane], TL)
    bb = pl.multiple_of(gv[2][lane], TL)
    ks = pl.ds(half * HK, HK)
    return [
        pltpu.async_copy(vui_hbm.at[ks, pl.ds(ub, TL)], buf.at[0], sems[phase]),
        pltpu.async_copy(viu_hbm.at[ks, pl.ds(ib, TL)], buf.at[1], sems[phase]),
        pltpu.async_copy(vil_hbm.at[ks, pl.ds(ib, TL)], buf.at[2], sems[phase]),
        pltpu.async_copy(vli_hbm.at[ks, pl.ds(bb, TL)], buf.at[3], sems[phase]),
    ]

  def compute_half(gv, s):
    lane = (s // 2) % G
    phase = s % NB
    buf = blk_v.at[phase]
    ci = jnp.full((L,), gv[3][lane], jnp.int32)
    cj = jnp.full((L,), gv[4][lane], jnp.int32)
    ck = jnp.full((L,), gv[5][lane], jnp.int32)
    p = None
    for q in range(HK // L):
      rows = q * L + kiota
      a0 = plsc.load_gather(buf.at[0], [rows, ci])
      b0 = plsc.load_gather(buf.at[1], [rows, cj])
      a1 = plsc.load_gather(buf.at[2], [rows, cj])
      b1 = plsc.load_gather(buf.at[3], [rows, ck])
      t = a0 * b0 + a1 * b1
      p = t if p is None else p + t
    return p

  def do_block(e0, nj):
    ns = 2 * nj
    gv_by_group = {}

    def get_gv(j):
      g = j // G
      if g not in gv_by_group:
        gv_by_group[g] = load_vecs(e0 + g * G)
      return gv_by_group[g]

    dq = [fire_sub(get_gv(s // 2), s) for s in range(DEPTH)]
    acc_el = None
    for s in range(ns):
      sf = s + DEPTH
      nd = fire_sub(get_gv(sf // 2), sf) if sf < ns else None
      for d in dq.pop(0):
        d.wait()
      p = compute_half(get_gv(s // 2), s)
      if s % 2 == 0:
        acc_el = p
      else:
        j = s // 2
        plsc.store_scatter(tr_v, [kiota * L + (j % G)], acc_el + p)
        if j % G == G - 1:
          acc = tr_v[pl.ds(0, L)]
          for l in range(1, L):
            acc += tr_v[pl.ds(l * L, L)]
          out_v[pl.ds(e0 + j - (G - 1), L)] = acc
      dq.append(nd)

  @pl.loop(0, BPW // BLK)
  def _blk(h):
    do_block(h * BLK, BLK)

  _tail = BPW - (BPW // BLK) * BLK
  if _tail:
    do_block((BPW // BLK) * BLK, _tail)

  pltpu.sync_copy(out_v, out_hbm.at[pl.ds(wid * BPW, BPW)])


@jax.jit
def _fpmc_sc(uid, basket_prev, iid, VIL, VLI, VUI, VIU):
  mesh = plsc.VectorSubcoreMesh(
      core_axis_name="c", subcore_axis_name="s", num_cores=NC, num_subcores=NS)
  run = pl.kernel(
      _fpmc_body,
      out_type=jax.ShapeDtypeStruct((BATCH,), jnp.float32),
      mesh=mesh,
      compiler_params=pltpu.CompilerParams(
          needs_layout_passes=False, use_tc_tiling_on_sc=True),
      scratch_types=[
          pltpu.VMEM((BPW,), jnp.int32),
          pltpu.VMEM((BPW,), jnp.int32),
          pltpu.VMEM((BPW,), jnp.int32),
          pltpu.VMEM((NB, 4, HK, TL), jnp.float32),
          pltpu.VMEM((BPW,), jnp.float32),
          pltpu.VMEM((L * L,), jnp.float32),
          pltpu.SemaphoreType.DMA,
          pltpu.SemaphoreType.DMA,
          pltpu.SemaphoreType.DMA,
          pltpu.SemaphoreType.DMA,
          pltpu.SemaphoreType.DMA,
          pltpu.SemaphoreType.DMA,
      ],
  )
  uid2 = uid.astype(jnp.int32).reshape(NW, BPW)
  bp2 = basket_prev.astype(jnp.int32).reshape(NW, BPW)
  iid2 = iid.astype(jnp.int32).reshape(NW, BPW)
  return run(uid2, bp2, iid2, VIL.T, VLI.T, VUI.T, VIU.T)


def kernel(uid, basket_prev, iid, VIL, VLI, VUI, VIU):
  return _fpmc_sc(uid, basket_prev, iid, VIL, VLI, VUI, VIU)


# speedup vs baseline: 2.4207x; 1.1836x over previous
import functools

import jax
import jax.numpy as jnp
from jax import lax
from jax.experimental import pallas as pl
from jax.experimental.pallas import tpu as pltpu
from jax.experimental.pallas import tpu_sc as plsc

N_USERS = 100000
N_ITEMS = 1000000
K = 64
BATCH = 16384

NC = 2
NS = 16
L = 16
NW = NC * NS
BPW = BATCH // NW
G = 16
NG = BPW // G
TL = 128


NB = 6
DEPTH = 4
HK = K // 2
BLK = 48
NBU = 4


def _fpmc_body(uid_hbm, bp_hbm, iid_hbm, vil_hbm, vli_hbm, vui_hbm, viu_hbm,
               out_hbm,
               uid_v, bp_v, iid_v, blk_v, ubuf_v, out_v, tr_v,
               sem0, sem1, sem2, sem3, sem4, sem5):
  wid = lax.axis_index("s") * NC + lax.axis_index("c")

  pltpu.sync_copy(uid_hbm.at[wid], uid_v)
  pltpu.sync_copy(bp_hbm.at[wid], bp_v)
  pltpu.sync_copy(iid_hbm.at[wid], iid_v)

  sems = [sem0, sem1, sem2, sem3, sem4, sem5]
  kiota = lax.iota(jnp.int32, L)

  def load_vecs(e):
    e = pl.multiple_of(e, L)
    u = uid_v[pl.ds(e, L)]
    i = iid_v[pl.ds(e, L)]
    b = bp_v[pl.ds(e, L)]
    return (u & ~7, i & ~(TL - 1), b & ~(TL - 1),
            u & 7, i & (TL - 1), b & (TL - 1))

  def fire_sub(gv, s):
    lane = (s // 2) % G
    half = s % 2
    phase = s % NB
    buf = blk_v.at[phase]
    ib = pl.multiple_of(gv[1][lane], TL)
    bb = pl.multiple_of(gv[2][lane], TL)
    ks = pl.ds(half * HK, HK)
    copies = [
        pltpu.async_copy(viu_hbm.at[ks, pl.ds(ib, TL)], buf.at[0], sems[phase]),
        pltpu.async_copy(vil_hbm.at[ks, pl.ds(ib, TL)], buf.at[1], sems[phase]),
        pltpu.async_copy(vli_hbm.at[ks, pl.ds(bb, TL)], buf.at[2], sems[phase]),
    ]
    if half == 0:
      ub = pl.multiple_of(gv[0][lane], 8)
      up = (s // 2) % NBU
      copies.append(
          pltpu.async_copy(vui_hbm.at[pl.ds(ub, 8), :], ubuf_v.at[up],
                           sems[phase]))
    return copies

  def compute_half(gv, s):
    lane = (s // 2) % G
    half = s % 2
    phase = s % NB
    up = (s // 2) % NBU
    buf = blk_v.at[phase]
    cu = jnp.full((L,), gv[3][lane], jnp.int32)
    cj = jnp.full((L,), gv[4][lane], jnp.int32)
    ck = jnp.full((L,), gv[5][lane], jnp.int32)
    p = None
    for q in range(HK // L):
      rows = q * L + kiota
      a0 = plsc.load_gather(ubuf_v.at[up], [cu, half * HK + rows])
      b0 = plsc.load_gather(buf.at[0], [rows, cj])
      a1 = plsc.load_gather(buf.at[1], [rows, cj])
      b1 = plsc.load_gather(buf.at[2], [rows, ck])
      t = a0 * b0 + a1 * b1
      p = t if p is None else p + t
    return p

  def do_block(e0, nj):
    ns = 2 * nj
    gv_by_group = {}

    def get_gv(j):
      g = j // G
      if g not in gv_by_group:
        gv_by_group[g] = load_vecs(e0 + g * G)
      return gv_by_group[g]

    dq = [fire_sub(get_gv(s // 2), s) for s in range(DEPTH)]
    acc_el = None
    for s in range(ns):
      sf = s + DEPTH
      nd = fire_sub(get_gv(sf // 2), sf) if sf < ns else None
      for d in dq.pop(0):
        d.wait()
      p = compute_half(get_gv(s // 2), s)
      if s % 2 == 0:
        acc_el = p
      else:
        j = s // 2
        plsc.store_scatter(tr_v, [kiota * L + (j % G)], acc_el + p)
        if j % G == G - 1:
          acc = tr_v[pl.ds(0, L)]
          for l in range(1, L):
            acc += tr_v[pl.ds(l * L, L)]
          out_v[pl.ds(e0 + j - (G - 1), L)] = acc
      dq.append(nd)

  @pl.loop(0, BPW // BLK)
  def _blk(h):
    do_block(h * BLK, BLK)

  _tail = BPW - (BPW // BLK) * BLK
  if _tail:
    do_block((BPW // BLK) * BLK, _tail)

  pltpu.sync_copy(out_v, out_hbm.at[pl.ds(wid * BPW, BPW)])


@jax.jit
def _fpmc_sc(uid, basket_prev, iid, VIL, VLI, VUI, VIU):
  mesh = plsc.VectorSubcoreMesh(
      core_axis_name="c", subcore_axis_name="s", num_cores=NC, num_subcores=NS)
  run = pl.kernel(
      _fpmc_body,
      out_type=jax.ShapeDtypeStruct((BATCH,), jnp.float32),
      mesh=mesh,
      compiler_params=pltpu.CompilerParams(
          needs_layout_passes=False, use_tc_tiling_on_sc=True),
      scratch_types=[
          pltpu.VMEM((BPW,), jnp.int32),
          pltpu.VMEM((BPW,), jnp.int32),
          pltpu.VMEM((BPW,), jnp.int32),
          pltpu.VMEM((NB, 3, HK, TL), jnp.float32),
          pltpu.VMEM((NBU, 8, K), jnp.float32),
          pltpu.VMEM((BPW,), jnp.float32),
          pltpu.VMEM((L * L,), jnp.float32),
          pltpu.SemaphoreType.DMA,
          pltpu.SemaphoreType.DMA,
          pltpu.SemaphoreType.DMA,
          pltpu.SemaphoreType.DMA,
          pltpu.SemaphoreType.DMA,
          pltpu.SemaphoreType.DMA,
      ],
  )
  uid2 = uid.astype(jnp.int32).reshape(NW, BPW)
  bp2 = basket_prev.astype(jnp.int32).reshape(NW, BPW)
  iid2 = iid.astype(jnp.int32).reshape(NW, BPW)
  return run(uid2, bp2, iid2, VIL.T, VLI.T, VUI, VIU.T)


def kernel(uid, basket_prev, iid, VIL, VLI, VUI, VIU):
  return _fpmc_sc(uid, basket_prev, iid, VIL, VLI, VUI, VIU)


# speedup vs baseline: 2.9559x; 1.2211x over previous
import functools

import jax
import jax.numpy as jnp
from jax import lax
from jax.experimental import pallas as pl
from jax.experimental.pallas import tpu as pltpu
from jax.experimental.pallas import tpu_sc as plsc

N_USERS = 100000
N_ITEMS = 1000000
K = 64
BATCH = 16384

NC = 2
NS = 16
L = 16
NW = NC * NS
BPW = BATCH // NW
G = 16
NG = BPW // G
TL = 128


NB = 6
DEPTH = 4
HK = K // 2
BLK = 48
NBU = 4


def _fpmc_body(uid_hbm, bp_hbm, iid_hbm, nt_hbm, ph_hbm,
               vil_hbm, vli_hbm, vui_hbm, viu_hbm,
               out_hbm,
               uid_v, bp_v, iid_v, nt_v, ph_v, blk_v, ij_v, ubuf_v, out_v, tr_v,
               sem0, sem1, sem2, sem3, sem4, sem5, ij_sem):
  wid = lax.axis_index("s") * NC + lax.axis_index("c")

  pltpu.sync_copy(uid_hbm.at[wid], uid_v)
  pltpu.sync_copy(bp_hbm.at[wid], bp_v)
  pltpu.sync_copy(iid_hbm.at[wid], iid_v)
  pltpu.sync_copy(nt_hbm.at[wid], nt_v)
  pltpu.sync_copy(ph_hbm.at[wid], ph_v)

  sems = [sem0, sem1, sem2, sem3, sem4, sem5]
  kiota = lax.iota(jnp.int32, L)

  def load_vecs(e):
    e = pl.multiple_of(e, L)
    u = uid_v[pl.ds(e, L)]
    i = iid_v[pl.ds(e, L)]
    b = bp_v[pl.ds(e, L)]
    return (u & ~7, i & ~(TL - 1), b & ~(TL - 1),
            u & 7, i & (TL - 1), b & (TL - 1),
            nt_v[pl.ds(e, L)], ph_v[pl.ds(e, L)])

  def fire_sub(gv, s):
    lane = (s // 2) % G
    half = s % 2
    phase = s % NB
    ib = pl.multiple_of(gv[1][lane], TL)
    bb = pl.multiple_of(gv[2][lane], TL)
    ks = pl.ds(half * HK, HK)
    copies = [
        pltpu.async_copy(vli_hbm.at[ks, pl.ds(bb, TL)], blk_v.at[phase],
                         sems[phase]),
    ]
    if half == 0:
      ub = pl.multiple_of(gv[0][lane], 8)
      up = (s // 2) % NBU
      copies.append(
          pltpu.async_copy(vui_hbm.at[pl.ds(ub, 8), :], ubuf_v.at[up],
                           sems[phase]))
    flag = gv[6][lane]
    p = gv[7][lane]

    @pl.when(flag == 1)
    def _():
      pltpu.async_copy(viu_hbm.at[ks, pl.ds(ib, TL)], ij_v.at[p, half, 0],
                       ij_sem)
      pltpu.async_copy(vil_hbm.at[ks, pl.ds(ib, TL)], ij_v.at[p, half, 1],
                       ij_sem)

    return copies

  def compute_half(gv, s):
    lane = (s // 2) % G
    half = s % 2
    phase = s % NB
    up = (s // 2) % NBU
    ib = pl.multiple_of(gv[1][lane], TL)
    ks = pl.ds(half * HK, HK)
    flag = gv[6][lane]
    p = gv[7][lane]

    @pl.when(flag == 1)
    def _():
      pltpu.make_async_copy(viu_hbm.at[ks, pl.ds(ib, TL)],
                            ij_v.at[p, half, 0], ij_sem).wait()
      pltpu.make_async_copy(vil_hbm.at[ks, pl.ds(ib, TL)],
                            ij_v.at[p, half, 1], ij_sem).wait()

    cu = jnp.full((L,), gv[3][lane], jnp.int32)
    cj = jnp.full((L,), gv[4][lane], jnp.int32)
    ck = jnp.full((L,), gv[5][lane], jnp.int32)
    pacc = None
    for q in range(HK // L):
      rows = q * L + kiota
      a0 = plsc.load_gather(ubuf_v.at[up], [cu, half * HK + rows])
      b0 = plsc.load_gather(ij_v.at[p, half, 0], [rows, cj])
      a1 = plsc.load_gather(ij_v.at[p, half, 1], [rows, cj])
      b1 = plsc.load_gather(blk_v.at[phase], [rows, ck])
      t = a0 * b0 + a1 * b1
      pacc = t if pacc is None else pacc + t
    return pacc

  def do_block(e0, nj):
    ns = 2 * nj
    gv_by_group = {}

    def get_gv(j):
      g = j // G
      if g not in gv_by_group:
        gv_by_group[g] = load_vecs(e0 + g * G)
      return gv_by_group[g]

    dq = [fire_sub(get_gv(s // 2), s) for s in range(DEPTH)]
    acc_el = None
    for s in range(ns):
      sf = s + DEPTH
      nd = fire_sub(get_gv(sf // 2), sf) if sf < ns else None
      for d in dq.pop(0):
        d.wait()
      p = compute_half(get_gv(s // 2), s)
      if s % 2 == 0:
        acc_el = p
      else:
        j = s // 2
        plsc.store_scatter(tr_v, [kiota * L + (j % G)], acc_el + p)
        if j % G == G - 1:
          acc = tr_v[pl.ds(0, L)]
          for l in range(1, L):
            acc += tr_v[pl.ds(l * L, L)]
          out_v[pl.ds(e0 + j - (G - 1), L)] = acc
      dq.append(nd)

  @pl.loop(0, BPW // BLK)
  def _blk(h):
    do_block(h * BLK, BLK)

  _tail = BPW - (BPW // BLK) * BLK
  if _tail:
    do_block((BPW // BLK) * BLK, _tail)

  pltpu.sync_copy(out_v, out_hbm.at[pl.ds(wid * BPW, BPW)])


@jax.jit
def _fpmc_sc(uid, basket_prev, iid, VIL, VLI, VUI, VIU):
  mesh = plsc.VectorSubcoreMesh(
      core_axis_name="c", subcore_axis_name="s", num_cores=NC, num_subcores=NS)
  run = pl.kernel(
      _fpmc_body,
      out_type=jax.ShapeDtypeStruct((BATCH,), jnp.float32),
      mesh=mesh,
      compiler_params=pltpu.CompilerParams(
          needs_layout_passes=False, use_tc_tiling_on_sc=True),
      scratch_types=[
          pltpu.VMEM((BPW,), jnp.int32),
          pltpu.VMEM((BPW,), jnp.int32),
          pltpu.VMEM((BPW,), jnp.int32),
          pltpu.VMEM((BPW,), jnp.int32),
          pltpu.VMEM((BPW,), jnp.int32),
          pltpu.VMEM((NB, HK, TL), jnp.float32),
          pltpu.VMEM((NB, 2, 2, HK, TL), jnp.float32),
          pltpu.VMEM((NBU, 8, K), jnp.float32),
          pltpu.VMEM((BPW,), jnp.float32),
          pltpu.VMEM((L * L,), jnp.float32),
          pltpu.SemaphoreType.DMA,
          pltpu.SemaphoreType.DMA,
          pltpu.SemaphoreType.DMA,
          pltpu.SemaphoreType.DMA,
          pltpu.SemaphoreType.DMA,
          pltpu.SemaphoreType.DMA,
          pltpu.SemaphoreType.DMA,
      ],
  )
  order = jnp.argsort(iid.astype(jnp.int32))
  suid = jnp.take(uid.astype(jnp.int32), order)
  sbp = jnp.take(basket_prev.astype(jnp.int32), order)
  siid = jnp.take(iid.astype(jnp.int32), order)
  tile = siid >> 7
  prev = jnp.concatenate([jnp.full((1,), -1, jnp.int32), tile[:-1]])
  first = (jnp.arange(BATCH, dtype=jnp.int32) % BPW) == 0
  newt = ((tile != prev) | first).astype(jnp.int32).reshape(NW, BPW)
  phase = ((jnp.cumsum(newt, axis=1) - 1) % NB).astype(jnp.int32)
  out_sorted = run(suid.reshape(NW, BPW), sbp.reshape(NW, BPW),
                   siid.reshape(NW, BPW), newt, phase,
                   VIL.T, VLI.T, VUI, VIU.T)
  return jnp.zeros((BATCH,), jnp.float32).at[order].set(out_sorted)


def kernel(uid, basket_prev, iid, VIL, VLI, VUI, VIU):
  return _fpmc_sc(uid, basket_prev, iid, VIL, VLI, VUI, VIU)
